# baseline (speedup 1.0000x reference)
import functools

import jax
import jax.numpy as jnp
from jax import lax
from jax.experimental import pallas as pl
from jax.experimental.pallas import tpu as pltpu
from jax.experimental.pallas import tpu_sc as plsc

N_EMB = 1000
ROWS = 4096 * 20
NC, NS, L = 2, 16, 16
NW = NC * NS
ROWS_PER_W = ROWS // NW
C = 64
CHUNK = C * N_EMB
NITER = ROWS_PER_W // C


def _one_hot_sc(idx_hbm, zeros_hbm):
    mesh = plsc.VectorSubcoreMesh(
        core_axis_name="c", subcore_axis_name="s", num_cores=NC, num_subcores=NS
    )

    @functools.partial(
        pl.kernel,
        out_type=jax.ShapeDtypeStruct((ROWS * N_EMB,), jnp.float32),
        mesh=mesh,
        scratch_types=[
            pltpu.VMEM((ROWS_PER_W,), jnp.int32),
            pltpu.VMEM((CHUNK,), jnp.float32),
            pltpu.VMEM((CHUNK,), jnp.float32),
            pltpu.SemaphoreType.DMA,
            pltpu.SemaphoreType.DMA,
        ],
        compiler_params=pltpu.CompilerParams(needs_layout_passes=False),
    )
    def body(idx_ref, zeros_ref, out_ref, idx_v, buf0, buf1, sem0, sem1):
        wid = lax.axis_index("s") * NC + lax.axis_index("c")
        row0 = wid * ROWS_PER_W

        pltpu.sync_copy(idx_ref.at[pl.ds(row0 * 1, ROWS_PER_W)], idx_v)
        pltpu.sync_copy(zeros_ref, buf0)
        pltpu.sync_copy(zeros_ref, buf1)

        lane = lax.iota(jnp.int32, L)
        ones_v = jnp.full((L,), 1.0, jnp.float32)
        zeros_v = jnp.zeros((L,), jnp.float32)

        def put(buf, j, val_v):
            for q in range(C // L):
                idx16 = idx_v[pl.ds(j * C + q * L, L)]
                pos = lane * N_EMB + (q * L * N_EMB) + idx16
                plsc.store_scatter(buf, [pos], val_v)

        def fire(buf, j, sem):
            base = (row0 + j * C) * N_EMB
            pltpu.async_copy(buf, out_ref.at[pl.ds(base, CHUNK)], sem)

        def drain(buf, sem):
            pltpu.make_async_copy(buf, out_ref.at[pl.ds(0, CHUNK)], sem).wait()

        bufs = (buf0, buf1)
        sems = (sem0, sem1)

        for b in range(2):
            put(bufs[b], b, ones_v)
            fire(bufs[b], b, sems[b])

        def step(g, carry):
            for b in range(2):
                j = 2 * g + b
                drain(bufs[b], sems[b])
                put(bufs[b], j - 2, zeros_v)
                put(bufs[b], j, ones_v)
                fire(bufs[b], j, sems[b])
            return carry

        lax.fori_loop(1, NITER // 2, step, 0)

        drain(buf0, sem0)
        drain(buf1, sem1)

    return body(idx_hbm, zeros_hbm)


@jax.jit
def kernel(inputs):
    idx = inputs.reshape(-1).astype(jnp.int32)
    zeros = jnp.zeros((CHUNK,), jnp.float32)
    flat = _one_hot_sc(idx, zeros)
    return flat.reshape(4096, 20, N_EMB)
